# baseline (speedup 1.0000x reference)
import functools

import jax
import jax.numpy as jnp
from jax import lax
from jax.experimental import pallas as pl
from jax.experimental.pallas import tpu as pltpu
from jax.experimental.pallas import tpu_sc as plsc

_SCALE = 8.0
_NBUF = 2
_BCHUNK = 128


@functools.lru_cache(maxsize=None)
def _make_kernel(batch, seq, vocab, d):
    info = plsc.get_sparse_core_info()
    nw = info.num_cores * info.num_subcores
    lanes = info.num_lanes
    assert d % lanes == 0 and batch % _BCHUNK == 0
    d_vecs = d // lanes
    n_bblk = batch // _BCHUNK
    n_tasks = seq * n_bblk
    assert n_tasks % (nw * _NBUF) == 0
    tpw = n_tasks // nw

    mesh = plsc.VectorSubcoreMesh(core_axis_name="c", subcore_axis_name="s")

    @functools.partial(
        pl.kernel,
        mesh=mesh,
        out_type=jax.ShapeDtypeStruct((seq, d, batch), jnp.float32),
        scratch_types=[
            [pltpu.VMEM((_BCHUNK,), jnp.int32) for _ in range(_NBUF)],
            [pltpu.VMEM((_BCHUNK,), jnp.int32) for _ in range(_NBUF)],
            [pltpu.VMEM((_BCHUNK, 2 * d), jnp.float32) for _ in range(_NBUF)],
            [pltpu.VMEM((d, _BCHUNK), jnp.float32) for _ in range(_NBUF)],
            [pltpu.SemaphoreType.DMA for _ in range(_NBUF)],
            [pltpu.SemaphoreType.DMA for _ in range(_NBUF)],
        ],
        compiler_params=pltpu.CompilerParams(needs_layout_passes=False),
    )
    def k(w2_hbm, xt_hbm, out_hbm, idxs, idx2s, supers, outs, gsems, ssems):
        wid = lax.axis_index("s") * info.num_cores + lax.axis_index("c")
        col = [lax.iota(jnp.int32, lanes) + dv * lanes for dv in range(d_vecs)]

        def decode(t):
            g = wid + t * nw
            return g // n_bblk, (g % n_bblk) * _BCHUNK

        def prep_and_gather(t, j):
            s, b0 = decode(t)
            pltpu.sync_copy(xt_hbm.at[s, pl.ds(b0, _BCHUNK)], idxs[j])
            def shift_body(i, carry):
                sl = pl.ds(i * lanes, lanes)
                idx2s[j][sl] = lax.shift_right_logical(idxs[j][sl], 1)
                return carry

            lax.fori_loop(0, _BCHUNK // lanes, shift_body, 0)
            pltpu.async_copy(w2_hbm.at[idx2s[j]], supers[j], gsems[j])

        def gather_wait(j):
            pltpu.make_async_copy(
                w2_hbm.at[pl.ds(0, _BCHUNK)], supers[j], gsems[j]
            ).wait()

        def store_start(t, j):
            s, b0 = decode(t)
            pltpu.async_copy(
                outs[j], out_hbm.at[s, :, pl.ds(b0, _BCHUNK)], ssems[j]
            )

        def store_wait(j):
            pltpu.make_async_copy(
                outs[j], out_hbm.at[0, :, pl.ds(0, _BCHUNK)], ssems[j]
            ).wait()

        def process(j):
            def row_body(b, carry):
                bcol = jnp.broadcast_to(b, (lanes,))
                idv = plsc.load_gather(idxs[j], [bcol])
                hv = (idv & 1) * d
                for dv in range(d_vecs):
                    v = plsc.load_gather(supers[j], [bcol, hv + col[dv]])
                    plsc.store_scatter(outs[j], [col[dv], bcol], v * _SCALE)
                return carry

            lax.fori_loop(0, _BCHUNK, row_body, 0)

        prep_and_gather(0, 0)

        def outer_body(p, carry):
            for j in range(_NBUF):
                t = p * _NBUF + j
                pt = t + _NBUF - 1
                pj = (j + _NBUF - 1) % _NBUF

                @pl.when(pt < tpw)
                def _():
                    prep_and_gather(pt, pj)

                gather_wait(j)
                @pl.when(t >= _NBUF)
                def _():
                    store_wait(j)

                process(j)
                store_start(t, j)
            return carry

        lax.fori_loop(0, tpw // _NBUF, outer_body, 0)
        for j in range(_NBUF):
            store_wait(j)

    return k


def kernel(x, embedding_weight):
    b, s = x.shape
    vocab, d = embedding_weight.shape
    w2 = embedding_weight.reshape(vocab // 2, 2 * d)
    xt = x.T
    k = _make_kernel(b, s, vocab, d)
    out = k(w2, xt)
    return jnp.transpose(out, (2, 0, 1))
